# baseline (speedup 1.0000x reference)
import jax
import jax.numpy as jnp
from jax.experimental import pallas as pl


def _mix_kernel(x_in_ref, x_out_ref, o_ref):
    o_ref[...] = 0.5 * (x_in_ref[...] + x_out_ref[...])


def kernel(x_in, x_out, edge_index):
    if x_in.shape[1] != x_out.shape[0]:
        return (x_in, x_out, edge_index)
    n, m = x_out.shape
    block_rows = 512
    mixed = pl.pallas_call(
        _mix_kernel,
        grid=(n // block_rows,),
        in_specs=[
            pl.BlockSpec((block_rows, m), lambda i: (i, 0)),
            pl.BlockSpec((block_rows, m), lambda i: (i, 0)),
        ],
        out_specs=pl.BlockSpec((block_rows, m), lambda i: (i, 0)),
        out_shape=jax.ShapeDtypeStruct((n, m), x_out.dtype),
    )(x_in, x_out)
    return (x_in, mixed, edge_index)
